# baseline (speedup 1.0000x reference)
import functools

import jax
import jax.numpy as jnp
from jax import lax
from jax.experimental import pallas as pl
from jax.experimental.pallas import tpu as pltpu
from jax.experimental.pallas import tpu_sc as plsc

B = 1024
C = 100000
K = 5
ALPHA = 1.0
TOPS = K + 1

SEG = 128
BLKW = 2048
NBLK = (C + BLKW - 1) // BLKW
SEG_PER_BLK = BLKW // SEG
NSEG = NBLK * SEG_PER_BLK

TROW = 128
TAB_ROWS = B * C // TROW

NW = 32
RPS = 2 * TOPS + 1
GROWS = B * RPS
ROWS_PER_W = GROWS // NW
CHUNK = 104
NCHUNK = ROWS_PER_W // CHUNK

NEG = float("-inf")


def _segmax_body(x_ref, o_ref):
    j = pl.program_id(0)

    def _maxes(xb):
        outs = [
            jnp.max(xb[:, s * SEG:(s + 1) * SEG], axis=1, keepdims=True)
            for s in range(SEG_PER_BLK)
        ]
        return jnp.concatenate(outs, axis=1)[None]

    @pl.when(j < NBLK - 1)
    def _full():
        o_ref[...] = _maxes(x_ref[...])

    @pl.when(j == NBLK - 1)
    def _tail():
        xb = x_ref[...]
        cols = j * BLKW + lax.broadcasted_iota(jnp.int32, (B, BLKW), 1)
        o_ref[...] = _maxes(jnp.where(cols < C, xb, NEG))


def _stage_segmax(x):
    out3 = pl.pallas_call(
        _segmax_body,
        grid=(NBLK,),
        in_specs=[pl.BlockSpec((B, BLKW), lambda j: (0, j))],
        out_specs=pl.BlockSpec((1, B, SEG_PER_BLK), lambda j: (j, 0, 0)),
        out_shape=jax.ShapeDtypeStruct((NBLK, B, SEG_PER_BLK), jnp.float32),
    )(x)
    return out3


def _top6_body(s_ref, o_ref):
    vals = s_ref[...]
    blk = lax.broadcasted_iota(jnp.int32, (NBLK, B, SEG_PER_BLK), 0)
    sg = lax.broadcasted_iota(jnp.int32, (NBLK, B, SEG_PER_BLK), 2)
    gid = blk * SEG_PER_BLK + sg
    big = jnp.int32(2**30)
    for t in range(TOPS):
        m = jnp.max(jnp.max(vals, axis=0), axis=1, keepdims=True)
        hit = vals == m[None]
        idx = jnp.min(
            jnp.min(jnp.where(hit, gid, big), axis=0), axis=1, keepdims=True
        )
        o_ref[:, t:t + 1] = idx
        vals = jnp.where(gid == idx[None], NEG, vals)
    o_ref[:, TOPS:8] = jnp.zeros((B, 8 - TOPS), jnp.int32)


def _stage_top6(segmax3):
    return pl.pallas_call(
        _top6_body,
        in_specs=[pl.BlockSpec((NBLK, B, SEG_PER_BLK), lambda: (0, 0, 0))],
        out_specs=pl.BlockSpec((B, 8), lambda: (0, 0)),
        out_shape=jax.ShapeDtypeStruct((B, 8), jnp.int32),
    )(segmax3)


@functools.cache
def _make_sc_gather():
    @functools.partial(
        pl.kernel,
        mesh=plsc.VectorSubcoreMesh(core_axis_name="c", subcore_axis_name="s"),
        out_type=jax.ShapeDtypeStruct((NW, ROWS_PER_W, TROW), jnp.float32),
        scratch_types=[
            pltpu.VMEM((NCHUNK, CHUNK), jnp.int32),
            pltpu.VMEM((ROWS_PER_W, TROW), jnp.float32),
            pltpu.SemaphoreType.DMA,
        ],
    )
    def gather_k(table_hbm, idx_hbm, out_hbm, idx_v, rows_v, sem):
        wid = lax.axis_index("s") * 2 + lax.axis_index("c")
        pltpu.sync_copy(idx_hbm.at[wid], idx_v)
        copies = [
            pltpu.async_copy(
                table_hbm.at[idx_v.at[c]],
                rows_v.at[pl.ds(c * CHUNK, CHUNK)],
                sem,
            )
            for c in range(NCHUNK)
        ]
        for cp in copies:
            cp.wait()
        pltpu.sync_copy(rows_v, out_hbm.at[wid])

    return gather_k


def _sc_gather(table, idx):
    return _make_sc_gather()(table, idx)


def _final_body(g_ref, x2_ref, ids_ref, y_ref, o_ref):
    y = y_ref[...]
    w = TOPS * SEG
    row = lax.broadcasted_iota(jnp.int32, (B, 1), 0)
    shift4 = jnp.bitwise_and(row, 3)
    cands = []
    for sh4 in range(4):
        cands.append(jnp.concatenate(
            [g_ref[:, t * 2 * SEG + sh4 * 32:t * 2 * SEG + sh4 * 32 + SEG]
             for t in range(TOPS)], axis=1))
    vals = cands[0]
    for sh4 in range(1, 4):
        vals = jnp.where(shift4 == sh4, cands[sh4], vals)
    iot = lax.broadcasted_iota(jnp.int32, (B, w), 1)
    loc = jnp.bitwise_and(iot, SEG - 1)
    seg = jnp.concatenate(
        [jnp.broadcast_to(ids_ref[:, t:t + 1], (B, SEG)) for t in range(TOPS)],
        axis=1,
    )
    col = seg * SEG + loc
    valid = (col < C) & (col != y)
    vals = jnp.where(valid, vals, NEG)
    big = jnp.int32(2**30)
    for _ in range(K - 1):
        m = jnp.max(vals, axis=1, keepdims=True)
        idx = jnp.min(jnp.where(vals == m, iot, big), axis=1, keepdims=True)
        vals = jnp.where(iot == idx, NEG, vals)
    tk = jnp.max(vals, axis=1, keepdims=True)
    lane2 = jnp.bitwise_and(shift4 * 32 + y, TROW - 1)
    l128 = lax.broadcasted_iota(jnp.int32, (B, TROW), 1)
    x2 = jnp.sum(
        jnp.where(l128 == lane2, x2_ref[...], 0.0),
        axis=1, keepdims=True,
    )
    loss = jnp.maximum(ALPHA + (tk - x2) * (1.0 / K), 0.0)
    o_ref[...] = jnp.sum(loss, keepdims=True)[:1, :1] * (1.0 / B)


def _stage_final(gmain, x2rows, ids, y2):
    return pl.pallas_call(
        _final_body,
        in_specs=[
            pl.BlockSpec((B, TOPS * 2 * SEG), lambda: (0, 0)),
            pl.BlockSpec((B, TROW), lambda: (0, 0)),
            pl.BlockSpec((B, 8), lambda: (0, 0)),
            pl.BlockSpec((B, 1), lambda: (0, 0)),
        ],
        out_specs=pl.BlockSpec((1, 1), lambda: (0, 0)),
        out_shape=jax.ShapeDtypeStruct((1, 1), jnp.float32),
    )(gmain, x2rows, ids, y2)


def kernel(x, y):
    segmax = _stage_segmax(x)
    ids8 = _stage_top6(segmax)
    ids = ids8[:, :TOPS]

    base_e = (jnp.arange(B, dtype=jnp.int32) * C)[:, None]
    r0 = (base_e + ids * SEG) // TROW
    seg_rows = r0[:, :, None] + jnp.arange(2, dtype=jnp.int32)
    y32 = y.astype(jnp.int32)
    x2_rows = (base_e[:, 0] + y32) // TROW
    idx_all = jnp.concatenate(
        [seg_rows.reshape(B, 2 * TOPS), x2_rows[:, None]], axis=1
    )
    idx_all = jnp.minimum(idx_all, TAB_ROWS - 1)
    idx_all = idx_all.reshape(NW, NCHUNK, CHUNK)

    table = x.reshape(TAB_ROWS, TROW)
    g = _sc_gather(table, idx_all).reshape(B, RPS, TROW)
    gmain = g[:, : 2 * TOPS].reshape(B, TOPS * 2 * SEG)
    x2rows = g[:, 2 * TOPS]

    out = _stage_final(gmain, x2rows, ids8, y32[:, None])
    return out[0, 0]


# speedup vs baseline: 1.9249x; 1.9249x over previous
import functools

import jax
import jax.numpy as jnp
from jax import lax
from jax.experimental import pallas as pl
from jax.experimental.pallas import tpu as pltpu
from jax.experimental.pallas import tpu_sc as plsc

B = 1024
C = 100000
K = 5
ALPHA = 1.0
TOPS = K + 1

SEG = 128
NSEG_REAL = (C + SEG - 1) // SEG
TAIL = NSEG_REAL - 1
TAILW = C - TAIL * SEG
NSEG = 784
AW = NSEG + TAILW
RB = 64
NRB = B // RB

NW = 32
SPW = B // NW
REQ = TOPS + 1
NTILE = SPW * REQ
HALF = NTILE // 2

FB = 128
FR = FB * REQ

NEG = float("-inf")


def _segmax_body(x_ref, o_ref):
    for s in range(NSEG_REAL - 1):
        o_ref[:, s:s + 1] = jnp.max(
            x_ref[:, s * SEG:(s + 1) * SEG], axis=1, keepdims=True
        )
    o_ref[:, TAIL:TAIL + 1] = jnp.max(
        x_ref[:, TAIL * SEG:C], axis=1, keepdims=True
    )
    o_ref[:, NSEG_REAL:NSEG] = jnp.full((RB, NSEG - NSEG_REAL), NEG,
                                        jnp.float32)
    o_ref[:, NSEG:AW] = x_ref[:, TAIL * SEG:C]


def _stage_segmax(x):
    return pl.pallas_call(
        _segmax_body,
        grid=(NRB,),
        in_specs=[pl.BlockSpec((RB, C), lambda j: (j, 0))],
        out_specs=pl.BlockSpec((RB, AW), lambda j: (j, 0)),
        out_shape=jax.ShapeDtypeStruct((B, AW), jnp.float32),
    )(x)


def _top6_body(s_ref, o_ref):
    vals = s_ref[:, :NSEG]
    iot = lax.broadcasted_iota(jnp.int32, (B, NSEG), 1)
    big = jnp.int32(2**30)
    for t in range(TOPS):
        m = jnp.max(vals, axis=1, keepdims=True)
        idx = jnp.min(jnp.where(vals == m, iot, big), axis=1, keepdims=True)
        o_ref[:, t:t + 1] = idx
        vals = jnp.where(iot == idx, NEG, vals)
    o_ref[:, TOPS:8] = jnp.zeros((B, 8 - TOPS), jnp.int32)


def _stage_top6(segmax):
    return pl.pallas_call(
        _top6_body,
        in_specs=[pl.BlockSpec((B, AW), lambda: (0, 0))],
        out_specs=pl.BlockSpec((B, 8), lambda: (0, 0)),
        out_shape=jax.ShapeDtypeStruct((B, 8), jnp.int32),
    )(segmax)


@functools.cache
def _make_sc_gather():
    @functools.partial(
        pl.kernel,
        mesh=plsc.VectorSubcoreMesh(core_axis_name="c", subcore_axis_name="s"),
        out_type=jax.ShapeDtypeStruct((NW, 4, 8, REQ, 8, SEG), jnp.float32),
        scratch_types=[
            pltpu.VMEM((SPW, 16), jnp.int32),
            pltpu.VMEM((2, 8, REQ, 8, SEG), jnp.float32),
            pltpu.SemaphoreType.DMA,
        ],
    )
    def gather_k(x_hbm, cols_hbm, out_hbm, cols_v, tiles_v, sem):
        wid = lax.axis_index("s") * 2 + lax.axis_index("c")
        pltpu.sync_copy(cols_hbm.at[wid], cols_v)
        for h in range(2):
            copies = []
            for sgl in range(2):
                sg = 2 * h + sgl
                for si in range(8):
                    s = sg * 8 + si
                    cvec = cols_v[s]
                    for j in range(REQ):
                        col = pl.multiple_of(cvec[j], SEG)
                        copies.append(pltpu.async_copy(
                            x_hbm.at[
                                pl.ds(pl.multiple_of(
                                    wid * SPW + sg * 8, 8), 8),
                                pl.ds(col, SEG)],
                            tiles_v.at[sgl, si, j],
                            sem,
                        ))
            for cp in copies:
                cp.wait()
            pltpu.sync_copy(tiles_v, out_hbm.at[wid, pl.ds(2 * h, 2)])

    return gather_k


def _sc_gather(x, cols):
    return _make_sc_gather()(x, cols)


GW = REQ * 8 * SEG


def _final_body(g_ref, tail_ref, ids_ref, y_ref, o_ref):
    c = pl.program_id(0)
    y = y_ref[...]
    g = g_ref[...]
    lane = lax.broadcasted_iota(jnp.int32, (FB, GW), 1)
    j_i = lane >> 10
    tr_i = jnp.bitwise_and(lane >> 7, 7)
    loc = jnp.bitwise_and(lane, SEG - 1)
    i8 = jnp.bitwise_and(
        lax.broadcasted_iota(jnp.int32, (FB, 1), 0), 7
    )
    segl = jnp.concatenate(
        [jnp.broadcast_to(ids_ref[:, t:t + 1], (FB, 8 * SEG))
         for t in range(TOPS)]
        + [jnp.zeros((FB, 8 * SEG), jnp.int32)], axis=1
    )
    col = segl * SEG + loc
    sel = tr_i == i8
    valid = sel & (j_i < TOPS) & (segl != TAIL) & (col != y)
    vals = jnp.where(valid, g, NEG)

    tail = tail_ref[...]
    l32 = lax.broadcasted_iota(jnp.int32, (FB, TAILW), 1)
    tvals = jnp.where(TAIL * SEG + l32 != y, tail, NEG)
    tvals = jnp.concatenate(
        [tvals, jnp.full((FB, SEG - TAILW), NEG, jnp.float32)], axis=1
    )
    vals = jnp.concatenate([vals, tvals], axis=1)

    pid = lax.broadcasted_iota(jnp.int32, (FB, GW + SEG), 1)
    big = jnp.int32(2**30)
    for _ in range(K - 1):
        m = jnp.max(vals, axis=1, keepdims=True)
        idx = jnp.min(jnp.where(vals == m, pid, big), axis=1, keepdims=True)
        vals = jnp.where(pid == idx, NEG, vals)
    tk = jnp.max(vals, axis=1, keepdims=True)

    x2a = jnp.sum(
        jnp.where(
            sel & (j_i == TOPS) & (loc == jnp.bitwise_and(y, SEG - 1)),
            g, 0.0),
        axis=1, keepdims=True,
    )
    x2b = jnp.sum(jnp.where(TAIL * SEG + l32 == y, tail, 0.0),
                  axis=1, keepdims=True)
    x2 = jnp.where(y >= TAIL * SEG, x2b, x2a)

    loss = jnp.maximum(ALPHA + (tk - x2) * (1.0 / K), 0.0)
    part = jnp.sum(loss, keepdims=True)[:1, :1] * (1.0 / B)

    @pl.when(c == 0)
    def _init():
        o_ref[...] = part

    @pl.when(c > 0)
    def _acc():
        o_ref[...] = o_ref[...] + part


def _stage_final(g2, tail32, ids, y2):
    nblk = B // FB
    return pl.pallas_call(
        _final_body,
        grid=(nblk,),
        in_specs=[
            pl.BlockSpec((FB, GW), lambda c: (c, 0)),
            pl.BlockSpec((FB, TAILW), lambda c: (c, 0)),
            pl.BlockSpec((FB, 8), lambda c: (c, 0)),
            pl.BlockSpec((FB, 1), lambda c: (c, 0)),
        ],
        out_specs=pl.BlockSpec((1, 1), lambda c: (0, 0)),
        out_shape=jax.ShapeDtypeStruct((1, 1), jnp.float32),
    )(g2, tail32, ids, y2)


def kernel(x, y):
    seg_out = _stage_segmax(x)
    ids8 = _stage_top6(seg_out)
    ids = ids8[:, :TOPS]
    tail32 = seg_out[:, NSEG:AW]

    y32 = y.astype(jnp.int32)
    segcol = jnp.where(ids == TAIL, 0, ids * SEG)
    x2col = jnp.where(y32 >= TAIL * SEG, 0, jnp.bitwise_and(y32, -SEG))
    cols = jnp.concatenate(
        [segcol, x2col[:, None], jnp.zeros((B, 9), jnp.int32)], axis=1
    )
    cols = cols.reshape(NW, SPW, 16)

    g = _sc_gather(x, cols)
    g2 = g.reshape(B, GW)

    out = _stage_final(g2, tail32, ids8, y32[:, None])
    return out[0, 0]


# speedup vs baseline: 3.4601x; 1.7975x over previous
import functools

import jax
import jax.numpy as jnp
from jax import lax
from jax.experimental import pallas as pl
from jax.experimental.pallas import tpu as pltpu
from jax.experimental.pallas import tpu_sc as plsc

B = 1024
C = 100000
K = 5
ALPHA = 1.0
TOPS = K + 1
REQ = TOPS + 1

STR = 8
NSTR = C // STR
CBLK = 4096
NCB = -(-C // CBLK)
SPB = CBLK // STR
NSTR_PAD = NCB * SPB

NW = 32
SPW = B // NW
HALFS = SPW // 2

FB = 128

NEG = float("-inf")


def _stripemax_body(x_ref, o_ref):
    j = pl.program_id(0)
    xb = x_ref[...]
    x3 = xb.reshape(SPB, STR, B)

    @pl.when(j < NCB - 1)
    def _full():
        o_ref[...] = jnp.max(x3, axis=1)

    @pl.when(j == NCB - 1)
    def _tail():
        r = (lax.broadcasted_iota(jnp.int32, (SPB, STR, B), 0) * STR
             + lax.broadcasted_iota(jnp.int32, (SPB, STR, B), 1))
        xm = jnp.where(j * CBLK + r < C, x3, NEG)
        o_ref[...] = jnp.max(xm, axis=1)


def _stage_stripemax(xt):
    return pl.pallas_call(
        _stripemax_body,
        grid=(NCB,),
        in_specs=[pl.BlockSpec((CBLK, B), lambda j: (j, 0))],
        out_specs=pl.BlockSpec((SPB, B), lambda j: (j, 0)),
        out_shape=jax.ShapeDtypeStruct((NSTR_PAD, B), jnp.float32),
    )(xt)


TB = 256


def _top6_body(s_ref, o_ref):
    vals = s_ref[...]
    iot = lax.broadcasted_iota(jnp.int32, (NSTR_PAD, TB), 0)
    big = jnp.int32(2**30)
    for t in range(TOPS):
        m = jnp.max(vals, axis=0, keepdims=True)
        idx = jnp.min(jnp.where(vals == m, iot, big), axis=0, keepdims=True)
        o_ref[t:t + 1, :] = idx
        vals = jnp.where(iot == idx, NEG, vals)
    o_ref[TOPS:8, :] = jnp.zeros((8 - TOPS, TB), jnp.int32)


def _stage_top6(stripemax):
    return pl.pallas_call(
        _top6_body,
        grid=(B // TB,),
        in_specs=[pl.BlockSpec((NSTR_PAD, TB), lambda c: (0, c))],
        out_specs=pl.BlockSpec((8, TB), lambda c: (0, c)),
        out_shape=jax.ShapeDtypeStruct((8, B), jnp.int32),
    )(stripemax)


@functools.cache
def _make_sc_gather():
    @functools.partial(
        pl.kernel,
        mesh=plsc.VectorSubcoreMesh(core_axis_name="c", subcore_axis_name="s"),
        out_type=jax.ShapeDtypeStruct((B, REQ, STR, 128), jnp.float32),
        scratch_types=[
            pltpu.VMEM((SPW, 16), jnp.int32),
            pltpu.VMEM((HALFS, REQ, STR, 128), jnp.float32),
            pltpu.SemaphoreType.DMA,
        ],
    )
    def gather_k(xt_hbm, rows_hbm, out_hbm, rows_v, tiles_v, sem):
        wid = lax.axis_index("s") * 2 + lax.axis_index("c")
        lane0 = pl.multiple_of((wid // 4) * 128, 128)
        pltpu.sync_copy(rows_hbm.at[wid], rows_v)
        for h in range(2):
            copies = []
            for sl in range(HALFS):
                s = h * HALFS + sl
                rvec = rows_v[s]
                for j in range(REQ):
                    r0 = pl.multiple_of(rvec[j], STR)
                    copies.append(pltpu.async_copy(
                        xt_hbm.at[pl.ds(r0, STR), pl.ds(lane0, 128)],
                        tiles_v.at[sl, j],
                        sem,
                    ))
            for cp in copies:
                cp.wait()
            pltpu.sync_copy(
                tiles_v,
                out_hbm.at[pl.ds(wid * SPW + h * HALFS, HALFS)],
            )

    return gather_k


def _sc_gather(xt, rows):
    return _make_sc_gather()(xt, rows)


def _final_body(g_ref, ids_ref, y_ref, o_ref):
    c = pl.program_id(0)
    y = y_ref[...]
    g = g_ref[...]
    sl = lax.broadcasted_iota(jnp.int32, (FB, 1, 1, 1), 0)
    lane = lax.broadcasted_iota(jnp.int32, (FB, REQ, STR, 128), 3)
    gd = jnp.max(jnp.where(lane == sl, g, NEG), axis=3)

    jdim = lax.broadcasted_iota(jnp.int32, (FB, REQ, STR), 1)
    row8 = lax.broadcasted_iota(jnp.int32, (FB, REQ, STR), 2)
    sj = jnp.concatenate(
        [jnp.broadcast_to(ids_ref[:, t:t + 1][:, :, None], (FB, 1, STR))
         for t in range(TOPS)]
        + [jnp.zeros((FB, 1, STR), jnp.int32)], axis=1
    )
    col = sj * STR + row8
    y3 = y[:, :, None]
    valid = (jdim < TOPS) & (col != y3)
    cand = jnp.where(valid, gd, NEG)

    pid = jdim * STR + row8
    big = jnp.int32(2**30)
    for _ in range(K - 1):
        m = jnp.max(jnp.max(cand, axis=2), axis=1)[:, None, None]
        idx = jnp.min(jnp.min(
            jnp.where(cand == m, pid, big), axis=2), axis=1)[:, None, None]
        cand = jnp.where(pid == idx, NEG, cand)
    tk = jnp.max(jnp.max(cand, axis=2), axis=1, keepdims=True)

    x2win = gd[:, TOPS, :]
    i8 = lax.broadcasted_iota(jnp.int32, (FB, STR), 1)
    x2 = jnp.sum(
        jnp.where(i8 == jnp.bitwise_and(y, STR - 1), x2win, 0.0),
        axis=1, keepdims=True,
    )

    loss = jnp.maximum(ALPHA + (tk - x2) * (1.0 / K), 0.0)
    part = jnp.sum(loss, keepdims=True)[:1, :1] * (1.0 / B)

    @pl.when(c == 0)
    def _init():
        o_ref[...] = part

    @pl.when(c > 0)
    def _acc():
        o_ref[...] = o_ref[...] + part


def _stage_final(g, ids, y2):
    nblk = B // FB
    return pl.pallas_call(
        _final_body,
        grid=(nblk,),
        in_specs=[
            pl.BlockSpec((FB, REQ, STR, 128), lambda c: (c, 0, 0, 0)),
            pl.BlockSpec((FB, 8), lambda c: (c, 0)),
            pl.BlockSpec((FB, 1), lambda c: (c, 0)),
        ],
        out_specs=pl.BlockSpec((1, 1), lambda c: (0, 0)),
        out_shape=jax.ShapeDtypeStruct((1, 1), jnp.float32),
    )(g, ids, y2)


def kernel(x, y):
    xt = x.T
    stripemax = _stage_stripemax(xt)
    idsT = _stage_top6(stripemax)
    ids = idsT.T

    y32 = y.astype(jnp.int32)
    rows = jnp.concatenate(
        [ids[:, :TOPS] * STR, jnp.bitwise_and(y32, -STR)[:, None],
         jnp.zeros((B, 9), jnp.int32)], axis=1
    )
    rows = rows.reshape(NW, SPW, 16)

    g = _sc_gather(xt, rows)

    out = _stage_final(g, ids, y32[:, None])
    return out[0, 0]
